# baseline (speedup 1.0000x reference)
import jax
import jax.numpy as jnp
from jax.experimental import pallas as pl

N = 10000
E = 320000
IN = 128
H = 8
D = 64
HID = 128


def _mm_body(h_ref, w0_ref, w1_ref, o0_ref, o1_ref):
    h = h_ref[...]
    o0_ref[...] = jnp.dot(h, w0_ref[...], preferred_element_type=jnp.float32)
    o1_ref[...] = jnp.dot(h, w1_ref[...], preferred_element_type=jnp.float32)


def _feats(h, W0, W1):
    BN = 2000
    grid = (N // BN,)
    return pl.pallas_call(
        _mm_body,
        grid=grid,
        in_specs=[
            pl.BlockSpec((BN, IN), lambda i: (i, 0)),
            pl.BlockSpec((IN, H * D), lambda i: (0, 0)),
            pl.BlockSpec((IN, H * D), lambda i: (0, 0)),
        ],
        out_specs=[
            pl.BlockSpec((BN, H * D), lambda i: (i, 0)),
            pl.BlockSpec((BN, H * D), lambda i: (i, 0)),
        ],
        out_shape=[
            jax.ShapeDtypeStruct((N, H * D), jnp.float32),
            jax.ShapeDtypeStruct((N, H * D), jnp.float32),
        ],
    )(h, W0, W1)


def _gat_edges(feat, edge_index, al, ar, b):
    feat = feat.reshape(N, H, D)
    src = edge_index[0]
    dst = edge_index[1]
    el = (feat * al[None, :, :]).sum(-1)
    er = (feat * ar[None, :, :]).sum(-1)
    e = jax.nn.leaky_relu(el[src] + er[dst], negative_slope=0.2)
    emax = jax.ops.segment_max(e, dst, num_segments=N)
    emax = jnp.where(jnp.isfinite(emax), emax, 0.0)
    ex = jnp.exp(e - emax[dst])
    den = jax.ops.segment_sum(ex, dst, num_segments=N)
    alpha = ex / (den[dst] + 1e-9)
    msg = feat[src] * alpha[:, :, None]
    out = jax.ops.segment_sum(msg, dst, num_segments=N)
    return jax.nn.elu(out + b[None, :, :]).reshape(N, H * D)


def kernel(h, edge_index_0, edge_index_1, W0, al0, ar0, b0, W1, al1, ar1, b1, pW1, pb1, pW2):
    feat0, feat1 = _feats(h, W0, W1)
    z0 = _gat_edges(feat0, edge_index_0, al0, ar0, b0)
    z1 = _gat_edges(feat1, edge_index_1, al1, ar1, b1)
    z = jnp.stack([z0, z1], axis=1)
    w = jnp.tanh(z @ pW1 + pb1) @ pW2
    beta = jax.nn.softmax(w, axis=1)
    return (beta * z).sum(1)


# speedup vs baseline: 20.8084x; 20.8084x over previous
import functools

import jax
import jax.numpy as jnp
from jax import lax
from jax.experimental import pallas as pl
from jax.experimental.pallas import tpu as pltpu
from jax.experimental.pallas import tpu_sc as plsc

N = 10000
E = 320000
IN = 128
H = 8
D = 64
HID = 128
P = 2

NT = 16
ER_ROWS = E // 128
NSA = 624


def _ka_body(h_ref, w0_ref, w1_ref, alm0_ref, arm0_ref, alm1_ref, arm1_ref,
             f0_ref, f1_ref, el0_ref, er0_ref, el1_ref, er1_ref,
             mel_ref, mer_ref):
    i = pl.program_id(0)
    hb = h_ref[...]
    f0 = jnp.dot(hb, w0_ref[...], preferred_element_type=jnp.float32)
    f1 = jnp.dot(hb, w1_ref[...], preferred_element_type=jnp.float32)
    f0_ref[...] = f0
    f1_ref[...] = f1
    el0 = jnp.dot(f0, alm0_ref[...], preferred_element_type=jnp.float32)
    er0 = jnp.dot(f0, arm0_ref[...], preferred_element_type=jnp.float32)
    el1 = jnp.dot(f1, alm1_ref[...], preferred_element_type=jnp.float32)
    er1 = jnp.dot(f1, arm1_ref[...], preferred_element_type=jnp.float32)
    el0_ref[...] = el0
    er0_ref[...] = er0
    el1_ref[...] = el1
    er1_ref[...] = er1

    @pl.when(i == 0)
    def _():
        mel_ref[...] = jnp.full((P, 8), -1e30, jnp.float32)
        mer_ref[...] = jnp.full((P, 8), -1e30, jnp.float32)

    mel_new = jnp.stack([jnp.full((8,), jnp.max(el0), jnp.float32),
                         jnp.full((8,), jnp.max(el1), jnp.float32)])
    mer_new = jnp.stack([jnp.full((8,), jnp.max(er0), jnp.float32),
                         jnp.full((8,), jnp.max(er1), jnp.float32)])
    mel_ref[...] = jnp.maximum(mel_ref[...], mel_new)
    mer_ref[...] = jnp.maximum(mer_ref[...], mer_new)


def _kernel_a(h, W0, W1, alm0, arm0, alm1, arm1):
    BN = 2000
    grid = (N // BN,)
    full = lambda i: (0, 0)
    return pl.pallas_call(
        _ka_body,
        grid=grid,
        in_specs=[
            pl.BlockSpec((BN, IN), lambda i: (i, 0)),
            pl.BlockSpec((IN, H * D), full),
            pl.BlockSpec((IN, H * D), full),
            pl.BlockSpec((H * D, 8), full),
            pl.BlockSpec((H * D, 8), full),
            pl.BlockSpec((H * D, 8), full),
            pl.BlockSpec((H * D, 8), full),
        ],
        out_specs=[
            pl.BlockSpec((BN, H * D), lambda i: (i, 0)),
            pl.BlockSpec((BN, H * D), lambda i: (i, 0)),
            pl.BlockSpec((BN, 8), lambda i: (i, 0)),
            pl.BlockSpec((BN, 8), lambda i: (i, 0)),
            pl.BlockSpec((BN, 8), lambda i: (i, 0)),
            pl.BlockSpec((BN, 8), lambda i: (i, 0)),
            pl.BlockSpec((P, 8), full),
            pl.BlockSpec((P, 8), full),
        ],
        out_shape=[
            jax.ShapeDtypeStruct((N, H * D), jnp.float32),
            jax.ShapeDtypeStruct((N, H * D), jnp.float32),
            jax.ShapeDtypeStruct((N, 8), jnp.float32),
            jax.ShapeDtypeStruct((N, 8), jnp.float32),
            jax.ShapeDtypeStruct((N, 8), jnp.float32),
            jax.ShapeDtypeStruct((N, 8), jnp.float32),
            jax.ShapeDtypeStruct((P, 8), jnp.float32),
            jax.ShapeDtypeStruct((P, 8), jnp.float32),
        ],
    )(h, W0, W1, alm0, arm0, alm1, arm1)


def _sc_body(featv, t1_hbm, t2_hbm, mm_hbm, edges_hbm,
             z_hbm, den_hbm, ex_hbm,
             t1_sh, t2_sh, den_sh, acc_sh,
             srcv, dstv, g1, g2, rows, mb, sem):
    p = lax.axis_index("c")
    s = lax.axis_index("s")
    r0 = s * NSA
    last = s == NT - 1

    pltpu.sync_copy(t1_hbm.at[p, pl.ds(r0, NSA)], t1_sh.at[pl.ds(r0, NSA)])
    pltpu.sync_copy(t2_hbm.at[p, pl.ds(r0, NSA)], t2_sh.at[pl.ds(r0, NSA)])
    pltpu.sync_copy(mm_hbm.at[p], mb)

    @pl.when(last)
    def _tail_stage():
        pltpu.sync_copy(t1_hbm.at[p, pl.ds(N - 16, 16)],
                        t1_sh.at[pl.ds(N - 16, 16)])
        pltpu.sync_copy(t2_hbm.at[p, pl.ds(N - 16, 16)],
                        t2_sh.at[pl.ds(N - 16, 16)])

    zvec = jnp.zeros((16,), jnp.float32)

    def _zero_rows():
        @pl.loop(0, 128)
        def _zr(b):
            for j in range(8):
                rows[b, pl.ds(16 * j, 16)] = zvec

    _zero_rows()

    for c in range(4):
        pltpu.sync_copy(rows.at[pl.ds(0, 128), pl.ds(0, 16)],
                        den_sh.at[pl.ds(r0 + 128 * c, 128)])
    pltpu.sync_copy(rows.at[pl.ds(0, 112), pl.ds(0, 16)],
                    den_sh.at[pl.ds(r0 + 512, 112)])

    @pl.when(last)
    def _tail_den0():
        pltpu.sync_copy(rows.at[pl.ds(0, 16), pl.ds(0, 16)],
                        den_sh.at[pl.ds(N - 16, 16)])

    m = mb[pl.ds(0, 16)][0]
    plsc.subcore_barrier()

    nrows = (ER_ROWS - 1 - s) // NT + 1

    @pl.loop(0, nrows)
    def _edge(i):
        r = s + NT * i
        pltpu.sync_copy(edges_hbm.at[p, 0, pl.ds(128 * r, 128)], srcv)
        pltpu.sync_copy(edges_hbm.at[p, 1, pl.ds(128 * r, 128)], dstv)
        pltpu.async_copy(t1_sh.at[srcv], g1, sem).wait()
        pltpu.async_copy(t2_sh.at[dstv], g2, sem).wait()

        @pl.loop(0, 128)
        def _cmp(b):
            v = g1[b, pl.ds(0, 16)] + g2[b, pl.ds(0, 16)]
            e = jnp.where(v > 0, v, 0.2 * v)
            g2[b, pl.ds(0, 16)] = jnp.exp(e - m)

        pltpu.sync_copy(g2, den_sh.at[dstv], add=True)
        pltpu.sync_copy(g2, ex_hbm.at[p, pl.ds(128 * r, 128)])

    plsc.subcore_barrier()

    pltpu.sync_copy(den_sh.at[pl.ds(r0, NSA)], den_hbm.at[p, pl.ds(r0, NSA)])

    @pl.when(last)
    def _tail_dend():
        pltpu.sync_copy(den_sh.at[pl.ds(N - 16, 16)],
                        den_hbm.at[p, pl.ds(N - 16, 16)])

    for k in range(4):
        _zero_rows()
        for c in range(4):
            pltpu.sync_copy(rows, acc_sh.at[pl.ds(r0 + 128 * c, 128)])
        pltpu.sync_copy(rows.at[pl.ds(0, 112)],
                        acc_sh.at[pl.ds(r0 + 512, 112)])

        @pl.when(last)
        def _tail_acc0():
            pltpu.sync_copy(rows.at[pl.ds(0, 16)],
                            acc_sh.at[pl.ds(N - 16, 16)])

        plsc.subcore_barrier()

        @pl.loop(0, nrows)
        def _agg(i):
            r = s + NT * i
            pltpu.sync_copy(edges_hbm.at[p, 0, pl.ds(128 * r, 128)], srcv)
            pltpu.sync_copy(edges_hbm.at[p, 1, pl.ds(128 * r, 128)], dstv)
            pltpu.sync_copy(ex_hbm.at[p, pl.ds(128 * r, 128)], g2)

            for j in range(8):
                sv = srcv[pl.ds(16 * j, 16)]
                srcv[pl.ds(16 * j, 16)] = (p * (4 * N)) + 4 * sv + k

            pltpu.async_copy(featv.at[srcv], rows, sem).wait()

            @pl.loop(0, 128)
            def _scale(b):
                exv = g2[b, pl.ds(0, 16)]
                a0 = exv[2 * k]
                a1 = exv[2 * k + 1]
                for j in range(4):
                    rows[b, pl.ds(16 * j, 16)] = rows[b, pl.ds(16 * j, 16)] * a0
                for j in range(4, 8):
                    rows[b, pl.ds(16 * j, 16)] = rows[b, pl.ds(16 * j, 16)] * a1

            pltpu.sync_copy(rows, acc_sh.at[dstv], add=True)

        plsc.subcore_barrier()
        pltpu.sync_copy(acc_sh.at[pl.ds(r0, NSA)],
                        z_hbm.at[p, pl.ds(r0, NSA), pl.ds(128 * k, 128)])

        @pl.when(last)
        def _tail_accd():
            pltpu.sync_copy(acc_sh.at[pl.ds(N - 16, 16)],
                            z_hbm.at[p, pl.ds(N - 16, 16), pl.ds(128 * k, 128)])

        plsc.subcore_barrier()


def _sc_kernel(featv, t1, t2, mm, edges):
    mesh = plsc.VectorSubcoreMesh(core_axis_name="c", subcore_axis_name="s")
    f = pl.kernel(
        _sc_body,
        out_type=[
            jax.ShapeDtypeStruct((P, N, 512), jnp.float32),
            jax.ShapeDtypeStruct((P, N, 16), jnp.float32),
            jax.ShapeDtypeStruct((P, E, 16), jnp.float32),
        ],
        mesh=mesh,
        compiler_params=pltpu.CompilerParams(use_tc_tiling_on_sc=False),
        scratch_types=[
            pltpu.VMEM_SHARED((N, 16), jnp.float32),
            pltpu.VMEM_SHARED((N, 16), jnp.float32),
            pltpu.VMEM_SHARED((N, 16), jnp.float32),
            pltpu.VMEM_SHARED((N, 128), jnp.float32),
            pltpu.VMEM((128,), jnp.int32),
            pltpu.VMEM((128,), jnp.int32),
            pltpu.VMEM((128, 16), jnp.float32),
            pltpu.VMEM((128, 16), jnp.float32),
            pltpu.VMEM((128, 128), jnp.float32),
            pltpu.VMEM((16,), jnp.float32),
            pltpu.SemaphoreType.DMA,
        ],
    )
    return f(featv, t1, t2, mm, edges)


def _kb_body(z_ref, den_ref, b0_ref, b1_ref, pw1_ref, pb1_ref, pw2_ref,
             out_ref):
    def path(pi, b_ref):
        zb = z_ref[pi]
        inv = 1.0 / jnp.maximum(den_ref[pi], 1e-30)
        parts = [zb[:, 64 * h:64 * (h + 1)] * inv[:, h:h + 1] for h in range(8)]
        zn = jnp.concatenate(parts, axis=1)
        x = zn + b_ref[...]
        za = jnp.where(x > 0, x, jnp.exp(jnp.minimum(x, 0.0)) - 1.0)
        t = jnp.tanh(jnp.dot(za, pw1_ref[...],
                             preferred_element_type=jnp.float32) + pb1_ref[...])
        w = jnp.sum(t * pw2_ref[...], axis=1, keepdims=True)
        return za, w

    z0, w0 = path(0, b0_ref)
    z1, w1 = path(1, b1_ref)
    wm = jnp.maximum(w0, w1)
    e0 = jnp.exp(w0 - wm)
    e1 = jnp.exp(w1 - wm)
    den = e0 + e1
    out_ref[...] = (e0 / den) * z0 + (e1 / den) * z1


def _kernel_b(z, den8, b0f, b1f, pW1, pb1, pW2t):
    BN = 1000
    grid = (N // BN,)
    full = lambda i: (0, 0)
    return pl.pallas_call(
        _kb_body,
        grid=grid,
        in_specs=[
            pl.BlockSpec((P, BN, 512), lambda i: (0, i, 0)),
            pl.BlockSpec((P, BN, 8), lambda i: (0, i, 0)),
            pl.BlockSpec((1, 512), full),
            pl.BlockSpec((1, 512), full),
            pl.BlockSpec((512, HID), full),
            pl.BlockSpec((1, HID), full),
            pl.BlockSpec((1, HID), full),
        ],
        out_specs=pl.BlockSpec((BN, 512), lambda i: (i, 0)),
        out_shape=jax.ShapeDtypeStruct((N, 512), jnp.float32),
    )(z, den8, b0f, b1f, pW1, pb1, pW2t)


def kernel(h, edge_index_0, edge_index_1, W0, al0, ar0, b0, W1, al1, ar1, b1,
           pW1, pb1, pW2):
    eye8 = jnp.eye(8, dtype=jnp.float32)
    expand = lambda a: (eye8[:, None, :] * a[:, :, None]).reshape(H * D, 8)
    alm0, arm0 = expand(al0), expand(ar0)
    alm1, arm1 = expand(al1), expand(ar1)

    f0, f1, el0, er0, el1, er1, mel, mer = _kernel_a(
        h, W0, W1, alm0, arm0, alm1, arm1)

    featv = jnp.concatenate(
        [f0.reshape(4 * N, 128), f1.reshape(4 * N, 128)], axis=0)
    el = jnp.stack([el0, el1])
    er = jnp.stack([er0, er1])
    t1 = jnp.concatenate([el, er], axis=-1)
    t2 = jnp.concatenate([er, el], axis=-1)
    sm = mel[:, :1] + mer[:, :1]
    mm = jnp.broadcast_to(jnp.maximum(sm, 0.2 * sm), (P, 16))

    edges = jnp.stack([
        jnp.stack([edge_index_0[0], edge_index_0[1]]),
        jnp.stack([edge_index_1[0], edge_index_1[1]]),
    ]).astype(jnp.int32)

    z, den, _ex = _sc_kernel(featv, t1, t2, mm, edges)

    den8 = den[:, :, :8]
    b0f = (b0.reshape(1, H * D)).astype(jnp.float32)
    b1f = (b1.reshape(1, H * D)).astype(jnp.float32)
    pb1r = pb1.reshape(1, HID)
    pW2t = pW2.reshape(1, HID)

    return _kernel_b(z, den8, b0f, b1f, pW1, pb1r, pW2t)
